# baseline (speedup 1.0000x reference)
import jax
import jax.numpy as jnp
from jax.experimental import pallas as pl

B, N, E, D = 8, 1250, 160000, 128
NUM_REL, NUM_WORD, L = 200, 20000, 20
NUM_INS, NUM_ITER, NUM_GNN = 2, 2, 2
BN = B * N


def _copy_body(x_ref, o_ref):
    o_ref[...] = x_ref[...]


def _pl_copy(x):
    return pl.pallas_call(
        _copy_body,
        out_shape=jax.ShapeDtypeStruct(x.shape, x.dtype),
    )(x)


def kernel(local_entity, query_entities, edge_head, edge_tail, edge_rel, query_text,
           seed_dist, answer_dist, word_emb, rel_emb, rel_emb_inv, W_rel, b_rel,
           W_type, b_type, ins_W, ins_b, W_gnn, W_self, b_gnn, w_score, w_att,
           W_reform, b_reform):
    rel_features = rel_emb @ W_rel + b_rel
    rel_features_inv = rel_emb_inv @ W_rel + b_rel
    ef = rel_features[edge_rel]
    ef_inv = rel_features_inv[edge_rel]
    agg = jnp.zeros((BN, D), dtype=jnp.float32).at[edge_tail].add(ef)
    agg = agg.at[edge_head].add(ef_inv)
    deg = jnp.zeros((BN,), dtype=jnp.float32).at[edge_tail].add(1.0).at[edge_head].add(1.0)
    h = jax.nn.relu((agg / jnp.maximum(deg, 1.0)[:, None]) @ W_type + b_type)
    q_mean = word_emb[query_text].mean(axis=1)
    ins = jnp.stack([jnp.tanh(q_mean @ ins_W[i] + ins_b[i]) for i in range(NUM_INS)], axis=1)
    curr = seed_dist / (jnp.sum(seed_dist, axis=1, keepdims=True) + 1e-8)
    batch_id = edge_head // N
    for t in range(NUM_ITER):
        for j in range(NUM_GNN):
            fact_prior = curr.reshape(BN)[edge_head]
            msgs = []
            for i in range(NUM_INS):
                ins_e = ins[:, i, :][batch_id]
                m = jax.nn.relu(ef * ins_e) * fact_prior[:, None]
                msgs.append(jnp.zeros((BN, D), dtype=jnp.float32).at[edge_tail].add(m))
            nb = jnp.concatenate(msgs, axis=1)
            h = jax.nn.relu(nb @ W_gnn[j] + h @ W_self[j] + b_gnn[j])
            curr = jax.nn.softmax((h @ w_score[j]).reshape(B, N), axis=1)
        att = jax.nn.softmax((h @ w_att[NUM_GNN - 1]).reshape(B, N), axis=1)
        hb = h.reshape(B, N, D)
        global_rep = jnp.einsum('bn,bnd->bd', att, hb)
        q_ent = jnp.einsum('bn,bnd->bd', query_entities, hb) / (jnp.sum(query_entities, axis=1, keepdims=True) + 1e-8)
        ins = jnp.stack([
            jnp.tanh(jnp.concatenate([ins[:, i, :], q_ent, global_rep], axis=1) @ W_reform[i] + b_reform[i])
            for i in range(NUM_INS)], axis=1)
    pred_dist = _pl_copy(curr)
    case_valid = (jnp.sum(answer_dist, axis=1, keepdims=True) > 0).astype(jnp.float32)
    tp_loss = -jnp.sum(answer_dist * jnp.log(pred_dist + 1e-8), axis=1, keepdims=True) * case_valid
    loss = jnp.sum(tp_loss) / B
    pred = jnp.argmax(pred_dist, axis=1)
    return (loss, pred, pred_dist)


# speedup vs baseline: 3.1890x; 3.1890x over previous
import functools

import jax
import jax.numpy as jnp
from jax import lax
from jax.experimental import pallas as pl
from jax.experimental.pallas import tpu as pltpu
from jax.experimental.pallas import tpu_sc as plsc

B, N, E, D = 8, 1250, 160000, 128
NUM_REL, NUM_WORD, L = 200, 20000, 20
NUM_INS, NUM_ITER, NUM_GNN = 2, 2, 2
BN = B * N

NC, NS = 2, 16
EPT = E // NS
CH = 80
NCHUNK = EPT // CH
ALN = 624
ZR = 208
REM = BN - NS * ALN
DIV_MAGIC, DIV_SHIFT = 6711, 23

_MESH = plsc.VectorSubcoreMesh(core_axis_name="c", subcore_axis_name="s",
                               num_cores=NC, num_subcores=NS)


def _zero_vmem(ref, nrows, ncols):
    z = jnp.zeros((16,), jnp.float32)

    def body(r, _):
        for g in range(ncols // 16):
            ref[r, pl.ds(g * 16, 16)] = z
        return 0

    lax.fori_loop(0, nrows, body, 0, unroll=4)


def _scatter_sc_body(dst_hbm, head_hbm, rel_hbm, t2_hbm, curr_hbm,
                     nb_hbm,
                     headb, dstb, relb, comb, priorb, rows, zbuf, currv, acc,
                     sem):
    cid = lax.axis_index("c")
    sid = lax.axis_index("s")

    pltpu.sync_copy(curr_hbm, currv)
    _zero_vmem(zbuf, ZR, D)
    for k in range(ALN // ZR):
        pltpu.sync_copy(zbuf, acc.at[pl.ds(sid * ALN + k * ZR, ZR)])

    @pl.when(sid == NS - 1)
    def _():
        pltpu.sync_copy(zbuf.at[pl.ds(0, REM)], acc.at[pl.ds(NS * ALN, REM)])

    plsc.subcore_barrier()

    def chunk(c, _):
        base = sid * EPT + c * CH
        pltpu.sync_copy(head_hbm.at[pl.ds(base, CH)], headb)
        pltpu.sync_copy(dst_hbm.at[pl.ds(cid * E + base, CH)], dstb)
        pltpu.sync_copy(rel_hbm.at[pl.ds(base, CH)], relb)

        def idxgrp(g, _):
            hv = headb[pl.ds(g * 16, 16)]
            rv = relb[pl.ds(g * 16, 16)]
            bv = jax.lax.shift_right_logical(hv * DIV_MAGIC, DIV_SHIFT)
            comb[pl.ds(g * 16, 16)] = cid * (NUM_REL * B) + bv * NUM_REL + rv
            priorb[pl.ds(g * 16, 16)] = plsc.load_gather(currv, [hv])
            return 0

        lax.fori_loop(0, CH // 16, idxgrp, 0, unroll=5)
        pltpu.async_copy(t2_hbm.at[comb], rows, sem).wait()

        def scale(r, _):
            pb = plsc.load_gather(priorb, [jnp.full((16,), r, jnp.int32)])
            for g in range(D // 16):
                rows[r, pl.ds(g * 16, 16)] = rows[r, pl.ds(g * 16, 16)] * pb
            return 0

        lax.fori_loop(0, CH, scale, 0, unroll=4)
        pltpu.sync_copy(rows, acc.at[dstb], add=True)
        return 0

    lax.fori_loop(0, NCHUNK, chunk, 0)
    plsc.subcore_barrier()
    pltpu.sync_copy(acc.at[pl.ds(sid * ALN, ALN)],
                    nb_hbm.at[pl.ds(cid * BN + sid * ALN, ALN)])

    @pl.when(sid == NS - 1)
    def _():
        pltpu.sync_copy(acc.at[pl.ds(NS * ALN, REM)],
                        nb_hbm.at[pl.ds(cid * BN + NS * ALN, REM)])


_scatter_sc = pl.kernel(
    _scatter_sc_body,
    out_type=jax.ShapeDtypeStruct((NC * BN, D), jnp.float32),
    mesh=_MESH,
    scratch_types=(
        pltpu.VMEM((CH,), jnp.int32),
        pltpu.VMEM((CH,), jnp.int32),
        pltpu.VMEM((CH,), jnp.int32),
        pltpu.VMEM((CH,), jnp.int32),
        pltpu.VMEM((CH,), jnp.float32),
        pltpu.VMEM((CH, D), jnp.float32),
        pltpu.VMEM((ZR, D), jnp.float32),
        pltpu.VMEM((BN,), jnp.float32),
        pltpu.VMEM_SHARED((BN, D), jnp.float32),
        pltpu.SemaphoreType.DMA,
    ),
    compiler_params=pltpu.CompilerParams(needs_layout_passes=False),
)


def _q_sc_body(qidx_hbm, wemb_hbm, qrows_hbm, qb0, qb1, r0, r1, sem):
    cid = lax.axis_index("c")
    sid = lax.axis_index("s")

    @pl.when(jnp.logical_and(cid == 0, sid == 0))
    def _():
        pltpu.sync_copy(qidx_hbm.at[pl.ds(0, 80)], qb0)
        pltpu.sync_copy(qidx_hbm.at[pl.ds(80, 80)], qb1)
        pltpu.async_copy(wemb_hbm.at[qb0], r0, sem).wait()
        pltpu.async_copy(wemb_hbm.at[qb1], r1, sem).wait()
        pltpu.sync_copy(r0, qrows_hbm.at[pl.ds(0, 80)])
        pltpu.sync_copy(r1, qrows_hbm.at[pl.ds(80, 80)])


_q_sc = pl.kernel(
    _q_sc_body,
    out_type=jax.ShapeDtypeStruct((B * L, D), jnp.float32),
    mesh=_MESH,
    scratch_types=(
        pltpu.VMEM((80,), jnp.int32),
        pltpu.VMEM((80,), jnp.int32),
        pltpu.VMEM((80, D), jnp.float32),
        pltpu.VMEM((80, D), jnp.float32),
        pltpu.SemaphoreType.DMA,
    ),
    compiler_params=pltpu.CompilerParams(needs_layout_passes=False),
)


def _mm(a, b):
    return jnp.dot(a, b, preferred_element_type=jnp.float32)


def _prep_body(re_ref, rei_ref, w_ref, b_ref, rf_ref, tt_ref):
    w = w_ref[...]
    bias = b_ref[...]
    rf = _mm(re_ref[...], w) + bias[None, :]
    rfi = _mm(rei_ref[...], w) + bias[None, :]
    rf_ref[...] = rf
    for b in range(B):
        tt_ref[pl.ds(b * NUM_REL, NUM_REL), :] = rf
        tt_ref[pl.ds((B + b) * NUM_REL, NUM_REL), :] = rfi


def _tc_prep(rel_emb, rel_emb_inv, W_rel, b_rel):
    return pl.pallas_call(
        _prep_body,
        out_shape=(
            jax.ShapeDtypeStruct((NUM_REL, D), jnp.float32),
            jax.ShapeDtypeStruct((NC * B * NUM_REL, D), jnp.float32),
        ),
    )(rel_emb, rel_emb_inv, W_rel, b_rel)


def _h0_body(agg_ref, deg_ref, w_ref, b_ref, h_ref):
    agg = agg_ref[0] + agg_ref[1]
    deg = deg_ref[0, :, 0] + deg_ref[1, :, 0]
    x = agg / jnp.maximum(deg, 1.0)[:, None]
    h_ref[...] = jax.nn.relu(_mm(x, w_ref[...]) + b_ref[...][None, :])


def _tc_h0(agg2, deg2, W_type, b_type):
    blk = 1000
    return pl.pallas_call(
        _h0_body,
        grid=(BN // blk,),
        in_specs=[
            pl.BlockSpec((NC, blk, D), lambda i: (0, i, 0)),
            pl.BlockSpec((NC, blk, D), lambda i: (0, i, 0)),
            pl.BlockSpec((D, D), lambda i: (0, 0)),
            pl.BlockSpec((D,), lambda i: (0,)),
        ],
        out_specs=pl.BlockSpec((blk, D), lambda i: (i, 0)),
        out_shape=jax.ShapeDtypeStruct((BN, D), jnp.float32),
    )(agg2, deg2, W_type, b_type)


def _ins0_body(qrows_ref, insW_ref, insb_ref, seed_ref, ins_ref, curr_ref):
    q_mean = jnp.mean(qrows_ref[...].reshape(B, L, D), axis=1)
    for i in range(NUM_INS):
        ins_ref[i] = jnp.tanh(_mm(q_mean, insW_ref[i]) + insb_ref[i][None, :])
    seed = seed_ref[...]
    curr_ref[...] = seed / (jnp.sum(seed, axis=1, keepdims=True) + 1e-8)


def _tc_ins0(qrows, ins_W, ins_b, seed_dist):
    return pl.pallas_call(
        _ins0_body,
        out_shape=(
            jax.ShapeDtypeStruct((NUM_INS, B, D), jnp.float32),
            jax.ShapeDtypeStruct((B, N), jnp.float32),
        ),
    )(qrows, ins_W, ins_b, seed_dist)


def _tbl_body(rf_ref, ins_ref, t2_ref):
    rf = rf_ref[...]
    for i in range(NUM_INS):
        ins_i = ins_ref[i]
        t = jax.nn.relu(rf[None, :, :] * ins_i[:, None, :])
        t2_ref[i] = t.reshape(B * NUM_REL, D)


def _tc_tables(rf, ins):
    return pl.pallas_call(
        _tbl_body,
        out_shape=jax.ShapeDtypeStruct((NUM_INS, B * NUM_REL, D), jnp.float32),
    )(rf, ins)


def _layer_body(nb_ref, h_ref, wg_ref, ws_ref, bg_ref, ho_ref):
    acc = _mm(nb_ref[0], wg_ref[0]) + _mm(nb_ref[1], wg_ref[1])
    acc += _mm(h_ref[...], ws_ref[...]) + bg_ref[...][None, :]
    ho_ref[...] = jax.nn.relu(acc)


def _tc_layer(nb2, h, Wg, Ws, bg):
    blk = 1000
    return pl.pallas_call(
        _layer_body,
        grid=(BN // blk,),
        in_specs=[
            pl.BlockSpec((NC, blk, D), lambda i: (0, i, 0)),
            pl.BlockSpec((blk, D), lambda i: (i, 0)),
            pl.BlockSpec((NC, D, D), lambda i: (0, 0, 0)),
            pl.BlockSpec((D, D), lambda i: (0, 0)),
            pl.BlockSpec((D,), lambda i: (0,)),
        ],
        out_specs=pl.BlockSpec((blk, D), lambda i: (i, 0)),
        out_shape=jax.ShapeDtypeStruct((BN, D), jnp.float32),
    )(nb2, h, Wg, Ws, bg)


def _softmax_lanes(s):
    m = jnp.max(s, axis=-1, keepdims=True)
    e = jnp.exp(s - m)
    return e / jnp.sum(e, axis=-1, keepdims=True)


def _score_body(hb_ref, w_ref, curr_ref):
    s = jnp.sum(hb_ref[...] * w_ref[...][None, None, :], axis=-1)
    curr_ref[...] = _softmax_lanes(s)


def _tc_score(hb, w):
    return pl.pallas_call(
        _score_body,
        out_shape=jax.ShapeDtypeStruct((B, N), jnp.float32),
    )(hb, w)


def _reform_body(hb_ref, watt_ref, qe_ref, ins_ref, wr_ref, br_ref, insn_ref):
    hb = hb_ref[...]
    satt = jnp.sum(hb * watt_ref[...][None, None, :], axis=-1)
    att = _softmax_lanes(satt)
    qe = qe_ref[...]
    global_rep = jnp.sum(att[:, :, None] * hb, axis=1)
    q_ent = jnp.sum(qe[:, :, None] * hb, axis=1)
    q_ent = q_ent / (jnp.sum(qe, axis=1, keepdims=True) + 1e-8)
    for i in range(NUM_INS):
        cat = jnp.concatenate([ins_ref[i], q_ent, global_rep], axis=1)
        insn_ref[i] = jnp.tanh(_mm(cat, wr_ref[i]) + br_ref[i][None, :])


def _tc_reform(hb, w_att_j, query_entities, ins, W_reform, b_reform):
    return pl.pallas_call(
        _reform_body,
        out_shape=jax.ShapeDtypeStruct((NUM_INS, B, D), jnp.float32),
    )(hb, w_att_j, query_entities, ins, W_reform, b_reform)


def _final_body(pd_ref, ans_ref, loss_ref, pred_ref):
    pd = pd_ref[...]
    ans = ans_ref[...]
    case_valid = (jnp.sum(ans, axis=1, keepdims=True) > 0).astype(jnp.float32)
    tp = -jnp.sum(ans * jnp.log(pd + 1e-8), axis=1, keepdims=True) * case_valid
    loss_ref[...] = (jnp.sum(tp) / B).reshape(1, 1)
    m = jnp.max(pd, axis=1, keepdims=True)
    idx = jax.lax.broadcasted_iota(jnp.int32, (B, N), 1)
    big = jnp.where(pd == m, idx, N)
    pred_ref[0, :] = jnp.min(big, axis=1)


def _tc_final(pred_dist, answer_dist):
    return pl.pallas_call(
        _final_body,
        out_shape=(
            jax.ShapeDtypeStruct((1, 1), jnp.float32),
            jax.ShapeDtypeStruct((1, B), jnp.int32),
        ),
    )(pred_dist, answer_dist)


def kernel(local_entity, query_entities, edge_head, edge_tail, edge_rel,
           query_text, seed_dist, answer_dist, word_emb, rel_emb, rel_emb_inv,
           W_rel, b_rel, W_type, b_type, ins_W, ins_b, W_gnn, W_self, b_gnn,
           w_score, w_att, W_reform, b_reform):
    edge_head = edge_head.astype(jnp.int32)
    edge_tail = edge_tail.astype(jnp.int32)
    edge_rel = edge_rel.astype(jnp.int32)

    rf, t_type = _tc_prep(rel_emb, rel_emb_inv, W_rel, b_rel)
    dst_th = jnp.concatenate([edge_tail, edge_head])
    dst_tt = jnp.concatenate([edge_tail, edge_tail])
    ones_bn = jnp.ones((BN,), jnp.float32)
    qidx = query_text.reshape(B * L).astype(jnp.int32)
    agg_f = _scatter_sc(dst_th, edge_head, edge_rel, t_type, ones_bn)
    ones_tbl = jnp.ones((NC * B * NUM_REL, D), jnp.float32)
    deg_f = _scatter_sc(dst_th, edge_head, edge_rel, ones_tbl, ones_bn)
    qrows = _q_sc(qidx, word_emb)
    agg2 = agg_f.reshape(NC, BN, D)
    deg2 = deg_f.reshape(NC, BN, D)
    h = _tc_h0(agg2, deg2, W_type, b_type)

    ins, curr = _tc_ins0(qrows, ins_W, ins_b, seed_dist)

    Wg_split = W_gnn.reshape(NUM_GNN, NUM_INS, D, D)
    for t in range(NUM_ITER):
        t2 = _tc_tables(rf, ins)
        t2_flat = t2.reshape(NUM_INS * B * NUM_REL, D)
        for j in range(NUM_GNN):
            nb_f = _scatter_sc(dst_tt, edge_head, edge_rel, t2_flat,
                               curr.reshape(BN))
            nb2 = nb_f.reshape(NC, BN, D)
            h = _tc_layer(nb2, h, Wg_split[j], W_self[j], b_gnn[j])
            curr = _tc_score(h.reshape(B, N, D), w_score[j])
        ins = _tc_reform(h.reshape(B, N, D), w_att[NUM_GNN - 1],
                         query_entities, ins, W_reform, b_reform)

    pred_dist = curr
    loss2, pred2 = _tc_final(pred_dist, answer_dist)
    loss = loss2.reshape(())
    pred = pred2.reshape(B)
    return (loss, pred, pred_dist)


# speedup vs baseline: 5.0916x; 1.5966x over previous
import functools

import jax
import jax.numpy as jnp
from jax import lax
from jax.experimental import pallas as pl
from jax.experimental.pallas import tpu as pltpu
from jax.experimental.pallas import tpu_sc as plsc

B, N, E, D = 8, 1250, 160000, 128
NUM_REL, NUM_WORD, L = 200, 20000, 20
NUM_INS, NUM_ITER, NUM_GNN = 2, 2, 2
BN = B * N

NC, NS = 2, 16
EPT = E // NS
CH = 80
NCHUNK = EPT // CH
ALN = 624
ZR = 208
REM = BN - NS * ALN
DIV_MAGIC, DIV_SHIFT = 6711, 23

_MESH = plsc.VectorSubcoreMesh(core_axis_name="c", subcore_axis_name="s",
                               num_cores=NC, num_subcores=NS)


def _zero_vmem(ref, nrows, ncols):
    z = jnp.zeros((16,), jnp.float32)

    def body(r, _):
        for g in range(ncols // 16):
            ref[r, pl.ds(g * 16, 16)] = z
        return 0

    lax.fori_loop(0, nrows, body, 0, unroll=4)


CH3 = 64
PAIR_E = 2 * CH3
NPAIR = EPT // PAIR_E
EREM = EPT - NPAIR * PAIR_E
PKW = 3 * PAIR_E
ZR2 = 52


def _scatter_sc_body(epack_hbm, erem_hbm, t2_hbm, curr_hbm,
                     nb_hbm,
                     ebuf, rbuf, comb0, comb1, dst0, dst1, prior0,
                     prior1, rows0, rows1, combr, dstr, priorr, rowsr, zbuf,
                     currv, acc, sg0, sg1, ss0, ss1):
    cid = lax.axis_index("c")
    sid = lax.axis_index("s")

    pltpu.sync_copy(curr_hbm, currv)
    _zero_vmem(zbuf, ZR2, D)
    for k in range(ALN // ZR2):
        pltpu.sync_copy(zbuf, acc.at[pl.ds(sid * ALN + k * ZR2, ZR2)])

    @pl.when(sid == NS - 1)
    def _():
        pltpu.sync_copy(zbuf.at[pl.ds(0, REM)], acc.at[pl.ds(NS * ALN, REM)])

    plsc.subcore_barrier()

    cbase = cid * (NUM_REL * B)
    tslot = cid * NS + sid

    def compute_idx(src, off0, n16, comb, dstc, prior):
        span = PKW // 3 if src is ebuf else EREM
        for g in range(n16):
            hv = src[pl.ds(off0 + g * 16, 16)]
            dv = src[pl.ds(span + off0 + g * 16, 16)]
            rv = src[pl.ds(2 * span + off0 + g * 16, 16)]
            bv = jax.lax.shift_right_logical(hv * DIV_MAGIC, DIV_SHIFT)
            comb[pl.ds(g * 16, 16)] = cbase + bv * NUM_REL + rv
            dstc[pl.ds(g * 16, 16)] = dv
            prior[pl.ds(g * 16, 16)] = plsc.load_gather(currv, [hv])

    def scale(nrows, rows, prior):
        def srow(r, _):
            pb = plsc.load_gather(prior, [jnp.full((16,), r, jnp.int32)])
            for g in range(D // 16):
                rows[r, pl.ds(g * 16, 16)] = rows[r, pl.ds(g * 16, 16)] * pb
            return 0

        lax.fori_loop(0, nrows, srow, 0, unroll=8)

    def pair(i, _):
        pltpu.sync_copy(epack_hbm.at[pl.ds((tslot * NPAIR + i) * PKW, PKW)],
                        ebuf)
        compute_idx(ebuf, 0, CH3 // 16, comb0, dst0, prior0)
        compute_idx(ebuf, CH3, CH3 // 16, comb1, dst1, prior1)
        g0 = pltpu.async_copy(t2_hbm.at[comb0], rows0, sg0)
        g1 = pltpu.async_copy(t2_hbm.at[comb1], rows1, sg1)
        g0.wait()
        scale(CH3, rows0, prior0)
        s0 = pltpu.async_copy(rows0, acc.at[dst0], ss0, add=True)
        g1.wait()
        scale(CH3, rows1, prior1)
        s1 = pltpu.async_copy(rows1, acc.at[dst1], ss1, add=True)
        s0.wait()
        s1.wait()
        return 0

    lax.fori_loop(0, NPAIR, pair, 0)

    pltpu.sync_copy(erem_hbm.at[pl.ds(tslot * (3 * EREM), 3 * EREM)], rbuf)
    compute_idx(rbuf, 0, EREM // 16, combr, dstr, priorr)
    pltpu.async_copy(t2_hbm.at[combr], rowsr, sg0).wait()
    scale(EREM, rowsr, priorr)
    pltpu.async_copy(rowsr, acc.at[dstr], ss0, add=True).wait()

    plsc.subcore_barrier()
    pltpu.sync_copy(acc.at[pl.ds(sid * ALN, ALN)],
                    nb_hbm.at[pl.ds(cid * BN + sid * ALN, ALN)])

    @pl.when(sid == NS - 1)
    def _():
        pltpu.sync_copy(acc.at[pl.ds(NS * ALN, REM)],
                        nb_hbm.at[pl.ds(cid * BN + NS * ALN, REM)])


_scatter_sc = pl.kernel(
    _scatter_sc_body,
    out_type=jax.ShapeDtypeStruct((NC * BN, D), jnp.float32),
    mesh=_MESH,
    scratch_types=(
        pltpu.VMEM((PKW,), jnp.int32),
        pltpu.VMEM((3 * EREM,), jnp.int32),
        pltpu.VMEM((CH3,), jnp.int32),
        pltpu.VMEM((CH3,), jnp.int32),
        pltpu.VMEM((CH3,), jnp.int32),
        pltpu.VMEM((CH3,), jnp.int32),
        pltpu.VMEM((CH3,), jnp.float32),
        pltpu.VMEM((CH3,), jnp.float32),
        pltpu.VMEM((CH3, D), jnp.float32),
        pltpu.VMEM((CH3, D), jnp.float32),
        pltpu.VMEM((EREM,), jnp.int32),
        pltpu.VMEM((EREM,), jnp.int32),
        pltpu.VMEM((EREM,), jnp.float32),
        pltpu.VMEM((EREM, D), jnp.float32),
        pltpu.VMEM((ZR2, D), jnp.float32),
        pltpu.VMEM((BN,), jnp.float32),
        pltpu.VMEM_SHARED((BN, D), jnp.float32),
        pltpu.SemaphoreType.DMA,
        pltpu.SemaphoreType.DMA,
        pltpu.SemaphoreType.DMA,
        pltpu.SemaphoreType.DMA,
    ),
    compiler_params=pltpu.CompilerParams(needs_layout_passes=False),
)


def _pack_edges(edge_head, edge_rel, dst_cat):
    h = edge_head.reshape(NS, EPT)
    r = edge_rel.reshape(NS, EPT)
    d = dst_cat.reshape(NC, NS, EPT)
    hm = jnp.broadcast_to(h[None, :, :NPAIR * PAIR_E].reshape(1, NS, NPAIR, PAIR_E),
                          (NC, NS, NPAIR, PAIR_E))
    rm = jnp.broadcast_to(r[None, :, :NPAIR * PAIR_E].reshape(1, NS, NPAIR, PAIR_E),
                          (NC, NS, NPAIR, PAIR_E))
    dm = d[:, :, :NPAIR * PAIR_E].reshape(NC, NS, NPAIR, PAIR_E)
    epack = jnp.concatenate([hm, dm, rm], axis=-1).reshape(-1)
    ht = jnp.broadcast_to(h[None, :, NPAIR * PAIR_E:], (NC, NS, EREM))
    rt = jnp.broadcast_to(r[None, :, NPAIR * PAIR_E:], (NC, NS, EREM))
    dt = d[:, :, NPAIR * PAIR_E:]
    erem = jnp.concatenate([ht, dt, rt], axis=-1).reshape(-1)
    return epack, erem


def _q_sc_body(qidx_hbm, wemb_hbm, qrows_hbm, qb0, qb1, r0, r1, sem):
    cid = lax.axis_index("c")
    sid = lax.axis_index("s")

    @pl.when(jnp.logical_and(cid == 0, sid == 0))
    def _():
        pltpu.sync_copy(qidx_hbm.at[pl.ds(0, 80)], qb0)
        pltpu.sync_copy(qidx_hbm.at[pl.ds(80, 80)], qb1)
        pltpu.async_copy(wemb_hbm.at[qb0], r0, sem).wait()
        pltpu.async_copy(wemb_hbm.at[qb1], r1, sem).wait()
        pltpu.sync_copy(r0, qrows_hbm.at[pl.ds(0, 80)])
        pltpu.sync_copy(r1, qrows_hbm.at[pl.ds(80, 80)])


_q_sc = pl.kernel(
    _q_sc_body,
    out_type=jax.ShapeDtypeStruct((B * L, D), jnp.float32),
    mesh=_MESH,
    scratch_types=(
        pltpu.VMEM((80,), jnp.int32),
        pltpu.VMEM((80,), jnp.int32),
        pltpu.VMEM((80, D), jnp.float32),
        pltpu.VMEM((80, D), jnp.float32),
        pltpu.SemaphoreType.DMA,
    ),
    compiler_params=pltpu.CompilerParams(needs_layout_passes=False),
)


def _mm(a, b):
    return jnp.dot(a, b, preferred_element_type=jnp.float32)


def _prep_body(re_ref, rei_ref, w_ref, b_ref, rf_ref, tt_ref):
    w = w_ref[...]
    bias = b_ref[...]
    rf = _mm(re_ref[...], w) + bias[None, :]
    rfi = _mm(rei_ref[...], w) + bias[None, :]
    rf_ref[...] = rf
    for b in range(B):
        tt_ref[pl.ds(b * NUM_REL, NUM_REL), :] = rf
        tt_ref[pl.ds((B + b) * NUM_REL, NUM_REL), :] = rfi


def _tc_prep(rel_emb, rel_emb_inv, W_rel, b_rel):
    return pl.pallas_call(
        _prep_body,
        out_shape=(
            jax.ShapeDtypeStruct((NUM_REL, D), jnp.float32),
            jax.ShapeDtypeStruct((NC * B * NUM_REL, D), jnp.float32),
        ),
    )(rel_emb, rel_emb_inv, W_rel, b_rel)


def _h0_body(agg_ref, deg_ref, w_ref, b_ref, h_ref):
    agg = agg_ref[0] + agg_ref[1]
    deg = deg_ref[0, :, 0] + deg_ref[1, :, 0]
    x = agg / jnp.maximum(deg, 1.0)[:, None]
    h_ref[...] = jax.nn.relu(_mm(x, w_ref[...]) + b_ref[...][None, :])


def _tc_h0(agg2, deg2, W_type, b_type):
    blk = 1000
    return pl.pallas_call(
        _h0_body,
        grid=(BN // blk,),
        in_specs=[
            pl.BlockSpec((NC, blk, D), lambda i: (0, i, 0)),
            pl.BlockSpec((NC, blk, D), lambda i: (0, i, 0)),
            pl.BlockSpec((D, D), lambda i: (0, 0)),
            pl.BlockSpec((D,), lambda i: (0,)),
        ],
        out_specs=pl.BlockSpec((blk, D), lambda i: (i, 0)),
        out_shape=jax.ShapeDtypeStruct((BN, D), jnp.float32),
    )(agg2, deg2, W_type, b_type)


def _ins0_body(qrows_ref, insW_ref, insb_ref, seed_ref, ins_ref, curr_ref):
    q_mean = jnp.mean(qrows_ref[...].reshape(B, L, D), axis=1)
    for i in range(NUM_INS):
        ins_ref[i] = jnp.tanh(_mm(q_mean, insW_ref[i]) + insb_ref[i][None, :])
    seed = seed_ref[...]
    curr_ref[...] = seed / (jnp.sum(seed, axis=1, keepdims=True) + 1e-8)


def _tc_ins0(qrows, ins_W, ins_b, seed_dist):
    return pl.pallas_call(
        _ins0_body,
        out_shape=(
            jax.ShapeDtypeStruct((NUM_INS, B, D), jnp.float32),
            jax.ShapeDtypeStruct((B, N), jnp.float32),
        ),
    )(qrows, ins_W, ins_b, seed_dist)


def _tbl_body(rf_ref, ins_ref, t2_ref):
    rf = rf_ref[...]
    for i in range(NUM_INS):
        ins_i = ins_ref[i]
        t = jax.nn.relu(rf[None, :, :] * ins_i[:, None, :])
        t2_ref[i] = t.reshape(B * NUM_REL, D)


def _tc_tables(rf, ins):
    return pl.pallas_call(
        _tbl_body,
        out_shape=jax.ShapeDtypeStruct((NUM_INS, B * NUM_REL, D), jnp.float32),
    )(rf, ins)


def _layer_body(nb_ref, h_ref, wg_ref, ws_ref, bg_ref, ho_ref):
    acc = _mm(nb_ref[0], wg_ref[0]) + _mm(nb_ref[1], wg_ref[1])
    acc += _mm(h_ref[...], ws_ref[...]) + bg_ref[...][None, :]
    ho_ref[...] = jax.nn.relu(acc)


def _tc_layer(nb2, h, Wg, Ws, bg):
    blk = 1000
    return pl.pallas_call(
        _layer_body,
        grid=(BN // blk,),
        in_specs=[
            pl.BlockSpec((NC, blk, D), lambda i: (0, i, 0)),
            pl.BlockSpec((blk, D), lambda i: (i, 0)),
            pl.BlockSpec((NC, D, D), lambda i: (0, 0, 0)),
            pl.BlockSpec((D, D), lambda i: (0, 0)),
            pl.BlockSpec((D,), lambda i: (0,)),
        ],
        out_specs=pl.BlockSpec((blk, D), lambda i: (i, 0)),
        out_shape=jax.ShapeDtypeStruct((BN, D), jnp.float32),
    )(nb2, h, Wg, Ws, bg)


def _softmax_lanes(s):
    m = jnp.max(s, axis=-1, keepdims=True)
    e = jnp.exp(s - m)
    return e / jnp.sum(e, axis=-1, keepdims=True)


def _score_body(hb_ref, w_ref, curr_ref):
    s = jnp.sum(hb_ref[...] * w_ref[...][None, None, :], axis=-1)
    curr_ref[...] = _softmax_lanes(s)


def _tc_score(hb, w):
    return pl.pallas_call(
        _score_body,
        out_shape=jax.ShapeDtypeStruct((B, N), jnp.float32),
    )(hb, w)


def _reform_body(hb_ref, watt_ref, qe_ref, ins_ref, wr_ref, br_ref, insn_ref):
    hb = hb_ref[...]
    satt = jnp.sum(hb * watt_ref[...][None, None, :], axis=-1)
    att = _softmax_lanes(satt)
    qe = qe_ref[...]
    global_rep = jnp.sum(att[:, :, None] * hb, axis=1)
    q_ent = jnp.sum(qe[:, :, None] * hb, axis=1)
    q_ent = q_ent / (jnp.sum(qe, axis=1, keepdims=True) + 1e-8)
    for i in range(NUM_INS):
        cat = jnp.concatenate([ins_ref[i], q_ent, global_rep], axis=1)
        insn_ref[i] = jnp.tanh(_mm(cat, wr_ref[i]) + br_ref[i][None, :])


def _tc_reform(hb, w_att_j, query_entities, ins, W_reform, b_reform):
    return pl.pallas_call(
        _reform_body,
        out_shape=jax.ShapeDtypeStruct((NUM_INS, B, D), jnp.float32),
    )(hb, w_att_j, query_entities, ins, W_reform, b_reform)


def _final_body(pd_ref, ans_ref, loss_ref, pred_ref):
    pd = pd_ref[...]
    ans = ans_ref[...]
    case_valid = (jnp.sum(ans, axis=1, keepdims=True) > 0).astype(jnp.float32)
    tp = -jnp.sum(ans * jnp.log(pd + 1e-8), axis=1, keepdims=True) * case_valid
    loss_ref[...] = (jnp.sum(tp) / B).reshape(1, 1)
    m = jnp.max(pd, axis=1, keepdims=True)
    idx = jax.lax.broadcasted_iota(jnp.int32, (B, N), 1)
    big = jnp.where(pd == m, idx, N)
    pred_ref[0, :] = jnp.min(big, axis=1)


def _tc_final(pred_dist, answer_dist):
    return pl.pallas_call(
        _final_body,
        out_shape=(
            jax.ShapeDtypeStruct((1, 1), jnp.float32),
            jax.ShapeDtypeStruct((1, B), jnp.int32),
        ),
    )(pred_dist, answer_dist)


def kernel(local_entity, query_entities, edge_head, edge_tail, edge_rel,
           query_text, seed_dist, answer_dist, word_emb, rel_emb, rel_emb_inv,
           W_rel, b_rel, W_type, b_type, ins_W, ins_b, W_gnn, W_self, b_gnn,
           w_score, w_att, W_reform, b_reform):
    edge_head = edge_head.astype(jnp.int32)
    edge_tail = edge_tail.astype(jnp.int32)
    edge_rel = edge_rel.astype(jnp.int32)

    rf, t_type = _tc_prep(rel_emb, rel_emb_inv, W_rel, b_rel)
    dst_th = jnp.concatenate([edge_tail, edge_head])
    dst_tt = jnp.concatenate([edge_tail, edge_tail])
    ones_bn = jnp.ones((BN,), jnp.float32)
    qidx = query_text.reshape(B * L).astype(jnp.int32)
    epack_th, erem_th = _pack_edges(edge_head, edge_rel, dst_th)
    epack_tt, erem_tt = _pack_edges(edge_head, edge_rel, dst_tt)
    agg_f = _scatter_sc(epack_th, erem_th, t_type, ones_bn)
    ones_tbl = jnp.ones((NC * B * NUM_REL, D), jnp.float32)
    dep1 = agg_f[0, 0] * 0.0
    deg_f = _scatter_sc(epack_th, erem_th, ones_tbl, ones_bn + dep1)
    qrows = _q_sc(qidx, word_emb)
    agg2 = agg_f.reshape(NC, BN, D)
    deg2 = deg_f.reshape(NC, BN, D)
    h = _tc_h0(agg2, deg2, W_type, b_type)

    ins, curr = _tc_ins0(qrows, ins_W, ins_b, seed_dist)
    curr = curr + deg_f[0, 0] * 0.0

    Wg_split = W_gnn.reshape(NUM_GNN, NUM_INS, D, D)
    for t in range(NUM_ITER):
        t2 = _tc_tables(rf, ins)
        t2_flat = t2.reshape(NUM_INS * B * NUM_REL, D)
        for j in range(NUM_GNN):
            nb_f = _scatter_sc(epack_tt, erem_tt, t2_flat, curr.reshape(BN))
            nb2 = nb_f.reshape(NC, BN, D)
            h = _tc_layer(nb2, h, Wg_split[j], W_self[j], b_gnn[j])
            curr = _tc_score(h.reshape(B, N, D), w_score[j])
        ins = _tc_reform(h.reshape(B, N, D), w_att[NUM_GNN - 1],
                         query_entities, ins, W_reform, b_reform)

    pred_dist = curr
    loss2, pred2 = _tc_final(pred_dist, answer_dist)
    loss = loss2.reshape(())
    pred = pred2.reshape(B)
    return (loss, pred, pred_dist)


# speedup vs baseline: 5.8593x; 1.1508x over previous
import functools

import jax
import jax.numpy as jnp
from jax import lax
from jax.experimental import pallas as pl
from jax.experimental.pallas import tpu as pltpu
from jax.experimental.pallas import tpu_sc as plsc

B, N, E, D = 8, 1250, 160000, 128
NUM_REL, NUM_WORD, L = 200, 20000, 20
NUM_INS, NUM_ITER, NUM_GNN = 2, 2, 2
BN = B * N

NC, NS = 2, 16
EPT = E // NS
CH = 80
NCHUNK = EPT // CH
ALN = 624
ZR = 208
REM = BN - NS * ALN
DIV_MAGIC, DIV_SHIFT = 6711, 23

_MESH = plsc.VectorSubcoreMesh(core_axis_name="c", subcore_axis_name="s",
                               num_cores=NC, num_subcores=NS)


def _zero_vmem(ref, nrows, ncols):
    z = jnp.zeros((16,), jnp.float32)

    def body(r, _):
        for g in range(ncols // 16):
            ref[r, pl.ds(g * 16, 16)] = z
        return 0

    lax.fori_loop(0, nrows, body, 0, unroll=4)


CH3 = 64
PAIR_E = 2 * CH3
NPAIR = EPT // PAIR_E
EREM = EPT - NPAIR * PAIR_E
PKW = 3 * PAIR_E
ZR2 = 52


def _scatter_sc_body(epack_hbm, erem_hbm, t2_hbm, curr_hbm,
                     nb_hbm,
                     ebufa, ebufb, rbuf, comb0, comb1, dst0, dst1, prior0,
                     prior1, rows0, rows1, combr, dstr, priorr, rowsr, zbuf,
                     currv, acc, sg0, sg1, ss0, ss1, sea, seb):
    cid = lax.axis_index("c")
    sid = lax.axis_index("s")

    pltpu.sync_copy(curr_hbm, currv)
    _zero_vmem(zbuf, ZR2, D)
    for k in range(ALN // ZR2):
        pltpu.sync_copy(zbuf, acc.at[pl.ds(sid * ALN + k * ZR2, ZR2)])

    @pl.when(sid == NS - 1)
    def _():
        pltpu.sync_copy(zbuf.at[pl.ds(0, REM)], acc.at[pl.ds(NS * ALN, REM)])

    plsc.subcore_barrier()

    cbase = cid * (NUM_REL * B)
    tslot = cid * NS + sid

    def compute_idx(src, off0, n16, span, comb, dstc, prior):
        for g in range(n16):
            cv = src[pl.ds(off0 + g * 16, 16)]
            dv = src[pl.ds(span + off0 + g * 16, 16)]
            hv = src[pl.ds(2 * span + off0 + g * 16, 16)]
            comb[pl.ds(g * 16, 16)] = cv + cbase
            dstc[pl.ds(g * 16, 16)] = dv
            prior[pl.ds(g * 16, 16)] = plsc.load_gather(currv, [hv])

    def scale(nrows, rows, prior):
        def srow(r, _):
            pb = plsc.load_gather(prior, [jnp.full((16,), r, jnp.int32)])
            for g in range(D // 16):
                rows[r, pl.ds(g * 16, 16)] = rows[r, pl.ds(g * 16, 16)] * pb
            return 0

        lax.fori_loop(0, nrows, srow, 0, unroll=8)

    def eload(p, buf, sem):
        return pltpu.async_copy(
            epack_hbm.at[pl.ds((tslot * NPAIR + p) * PKW, PKW)], buf, sem)

    def process(src):
        compute_idx(src, 0, CH3 // 16, PAIR_E, comb0, dst0, prior0)
        g0 = pltpu.async_copy(t2_hbm.at[comb0], rows0, sg0)
        compute_idx(src, CH3, CH3 // 16, PAIR_E, comb1, dst1, prior1)
        g1 = pltpu.async_copy(t2_hbm.at[comb1], rows1, sg1)
        g0.wait()
        scale(CH3, rows0, prior0)
        s0 = pltpu.async_copy(rows0, acc.at[dst0], ss0, add=True)
        g1.wait()
        scale(CH3, rows1, prior1)
        s1 = pltpu.async_copy(rows1, acc.at[dst1], ss1, add=True)
        return s0, s1

    eload(0, ebufa, sea)

    def two_pairs(k, _):
        pltpu.make_async_copy(epack_hbm.at[pl.ds(0, PKW)], ebufa, sea).wait()
        eb = eload(2 * k + 1, ebufb, seb)
        s0, s1 = process(ebufa)
        eb.wait()
        s0.wait()
        s1.wait()
        eload(jnp.minimum(2 * k + 2, NPAIR - 1), ebufa, sea)
        s0b, s1b = process(ebufb)
        s0b.wait()
        s1b.wait()
        return 0

    lax.fori_loop(0, NPAIR // 2, two_pairs, 0)
    pltpu.make_async_copy(epack_hbm.at[pl.ds(0, PKW)], ebufa, sea).wait()

    pltpu.sync_copy(erem_hbm.at[pl.ds(tslot * (3 * EREM), 3 * EREM)], rbuf)
    compute_idx(rbuf, 0, EREM // 16, EREM, combr, dstr, priorr)
    pltpu.async_copy(t2_hbm.at[combr], rowsr, sg0).wait()
    scale(EREM, rowsr, priorr)
    pltpu.async_copy(rowsr, acc.at[dstr], ss0, add=True).wait()

    plsc.subcore_barrier()
    pltpu.sync_copy(acc.at[pl.ds(sid * ALN, ALN)],
                    nb_hbm.at[pl.ds(cid * BN + sid * ALN, ALN)])

    @pl.when(sid == NS - 1)
    def _():
        pltpu.sync_copy(acc.at[pl.ds(NS * ALN, REM)],
                        nb_hbm.at[pl.ds(cid * BN + NS * ALN, REM)])


_scatter_sc = pl.kernel(
    _scatter_sc_body,
    out_type=jax.ShapeDtypeStruct((NC * BN, D), jnp.float32),
    mesh=_MESH,
    scratch_types=(
        pltpu.VMEM((PKW,), jnp.int32),
        pltpu.VMEM((PKW,), jnp.int32),
        pltpu.VMEM((3 * EREM,), jnp.int32),
        pltpu.VMEM((CH3,), jnp.int32),
        pltpu.VMEM((CH3,), jnp.int32),
        pltpu.VMEM((CH3,), jnp.int32),
        pltpu.VMEM((CH3,), jnp.int32),
        pltpu.VMEM((CH3,), jnp.float32),
        pltpu.VMEM((CH3,), jnp.float32),
        pltpu.VMEM((CH3, D), jnp.float32),
        pltpu.VMEM((CH3, D), jnp.float32),
        pltpu.VMEM((EREM,), jnp.int32),
        pltpu.VMEM((EREM,), jnp.int32),
        pltpu.VMEM((EREM,), jnp.float32),
        pltpu.VMEM((EREM, D), jnp.float32),
        pltpu.VMEM((ZR2, D), jnp.float32),
        pltpu.VMEM((BN,), jnp.float32),
        pltpu.VMEM_SHARED((BN, D), jnp.float32),
        pltpu.SemaphoreType.DMA,
        pltpu.SemaphoreType.DMA,
        pltpu.SemaphoreType.DMA,
        pltpu.SemaphoreType.DMA,
        pltpu.SemaphoreType.DMA,
        pltpu.SemaphoreType.DMA,
    ),
    compiler_params=pltpu.CompilerParams(needs_layout_passes=False),
)


def _combo_body(h_ref, r_ref, c_ref):
    b = jax.lax.shift_right_logical(h_ref[...] * DIV_MAGIC, DIV_SHIFT)
    c_ref[...] = b * NUM_REL + r_ref[...]


def _tc_combo(edge_head, edge_rel):
    out = pl.pallas_call(
        _combo_body,
        out_shape=jax.ShapeDtypeStruct((E // 128, 128), jnp.int32),
    )(edge_head.reshape(E // 128, 128), edge_rel.reshape(E // 128, 128))
    return out.reshape(E)


def _pack_edges(edge_combo, dst_cat, edge_head):
    c = edge_combo.reshape(NS, EPT)
    h = edge_head.reshape(NS, EPT)
    d = dst_cat.reshape(NC, NS, EPT)
    nmain = NPAIR * PAIR_E
    cm = jnp.broadcast_to(c[None, :, :nmain].reshape(1, NS, NPAIR, PAIR_E),
                          (NC, NS, NPAIR, PAIR_E))
    hm = jnp.broadcast_to(h[None, :, :nmain].reshape(1, NS, NPAIR, PAIR_E),
                          (NC, NS, NPAIR, PAIR_E))
    dm = d[:, :, :nmain].reshape(NC, NS, NPAIR, PAIR_E)
    epack = jnp.concatenate([cm, dm, hm], axis=-1).reshape(-1)
    ct = jnp.broadcast_to(c[None, :, nmain:], (NC, NS, EREM))
    ht = jnp.broadcast_to(h[None, :, nmain:], (NC, NS, EREM))
    dt = d[:, :, nmain:]
    erem = jnp.concatenate([ct, dt, ht], axis=-1).reshape(-1)
    return epack, erem


def _q_sc_body(qidx_hbm, wemb_hbm, qrows_hbm, qb0, qb1, r0, r1, sem):
    cid = lax.axis_index("c")
    sid = lax.axis_index("s")

    @pl.when(jnp.logical_and(cid == 0, sid == 0))
    def _():
        pltpu.sync_copy(qidx_hbm.at[pl.ds(0, 80)], qb0)
        pltpu.sync_copy(qidx_hbm.at[pl.ds(80, 80)], qb1)
        pltpu.async_copy(wemb_hbm.at[qb0], r0, sem).wait()
        pltpu.async_copy(wemb_hbm.at[qb1], r1, sem).wait()
        pltpu.sync_copy(r0, qrows_hbm.at[pl.ds(0, 80)])
        pltpu.sync_copy(r1, qrows_hbm.at[pl.ds(80, 80)])


_q_sc = pl.kernel(
    _q_sc_body,
    out_type=jax.ShapeDtypeStruct((B * L, D), jnp.float32),
    mesh=_MESH,
    scratch_types=(
        pltpu.VMEM((80,), jnp.int32),
        pltpu.VMEM((80,), jnp.int32),
        pltpu.VMEM((80, D), jnp.float32),
        pltpu.VMEM((80, D), jnp.float32),
        pltpu.SemaphoreType.DMA,
    ),
    compiler_params=pltpu.CompilerParams(needs_layout_passes=False),
)


def _mm(a, b):
    return jnp.dot(a, b, preferred_element_type=jnp.float32)


def _prep_body(re_ref, rei_ref, w_ref, b_ref, rf_ref, tt_ref):
    w = w_ref[...]
    bias = b_ref[...]
    rf = _mm(re_ref[...], w) + bias[None, :]
    rfi = _mm(rei_ref[...], w) + bias[None, :]
    rf_ref[...] = rf
    for b in range(B):
        tt_ref[pl.ds(b * NUM_REL, NUM_REL), :] = rf
        tt_ref[pl.ds((B + b) * NUM_REL, NUM_REL), :] = rfi


def _tc_prep(rel_emb, rel_emb_inv, W_rel, b_rel):
    return pl.pallas_call(
        _prep_body,
        out_shape=(
            jax.ShapeDtypeStruct((NUM_REL, D), jnp.float32),
            jax.ShapeDtypeStruct((NC * B * NUM_REL, D), jnp.float32),
        ),
    )(rel_emb, rel_emb_inv, W_rel, b_rel)


def _h0_body(agg_ref, deg_ref, w_ref, b_ref, h_ref):
    agg = agg_ref[0] + agg_ref[1]
    deg = deg_ref[0, :, 0] + deg_ref[1, :, 0]
    x = agg / jnp.maximum(deg, 1.0)[:, None]
    h_ref[...] = jax.nn.relu(_mm(x, w_ref[...]) + b_ref[...][None, :])


def _tc_h0(agg2, deg2, W_type, b_type):
    blk = 1000
    return pl.pallas_call(
        _h0_body,
        grid=(BN // blk,),
        in_specs=[
            pl.BlockSpec((NC, blk, D), lambda i: (0, i, 0)),
            pl.BlockSpec((NC, blk, D), lambda i: (0, i, 0)),
            pl.BlockSpec((D, D), lambda i: (0, 0)),
            pl.BlockSpec((D,), lambda i: (0,)),
        ],
        out_specs=pl.BlockSpec((blk, D), lambda i: (i, 0)),
        out_shape=jax.ShapeDtypeStruct((BN, D), jnp.float32),
    )(agg2, deg2, W_type, b_type)


def _ins0_body(qrows_ref, insW_ref, insb_ref, seed_ref, ins_ref, curr_ref):
    q_mean = jnp.mean(qrows_ref[...].reshape(B, L, D), axis=1)
    for i in range(NUM_INS):
        ins_ref[i] = jnp.tanh(_mm(q_mean, insW_ref[i]) + insb_ref[i][None, :])
    seed = seed_ref[...]
    curr_ref[...] = seed / (jnp.sum(seed, axis=1, keepdims=True) + 1e-8)


def _tc_ins0(qrows, ins_W, ins_b, seed_dist):
    return pl.pallas_call(
        _ins0_body,
        out_shape=(
            jax.ShapeDtypeStruct((NUM_INS, B, D), jnp.float32),
            jax.ShapeDtypeStruct((B, N), jnp.float32),
        ),
    )(qrows, ins_W, ins_b, seed_dist)


def _tbl_body(rf_ref, ins_ref, t2_ref):
    rf = rf_ref[...]
    for i in range(NUM_INS):
        ins_i = ins_ref[i]
        t = jax.nn.relu(rf[None, :, :] * ins_i[:, None, :])
        t2_ref[i] = t.reshape(B * NUM_REL, D)


def _tc_tables(rf, ins):
    return pl.pallas_call(
        _tbl_body,
        out_shape=jax.ShapeDtypeStruct((NUM_INS, B * NUM_REL, D), jnp.float32),
    )(rf, ins)


def _layer_body(nb_ref, h_ref, wg_ref, ws_ref, bg_ref, ho_ref):
    acc = _mm(nb_ref[0], wg_ref[0]) + _mm(nb_ref[1], wg_ref[1])
    acc += _mm(h_ref[...], ws_ref[...]) + bg_ref[...][None, :]
    ho_ref[...] = jax.nn.relu(acc)


def _tc_layer(nb2, h, Wg, Ws, bg):
    blk = 1000
    return pl.pallas_call(
        _layer_body,
        grid=(BN // blk,),
        in_specs=[
            pl.BlockSpec((NC, blk, D), lambda i: (0, i, 0)),
            pl.BlockSpec((blk, D), lambda i: (i, 0)),
            pl.BlockSpec((NC, D, D), lambda i: (0, 0, 0)),
            pl.BlockSpec((D, D), lambda i: (0, 0)),
            pl.BlockSpec((D,), lambda i: (0,)),
        ],
        out_specs=pl.BlockSpec((blk, D), lambda i: (i, 0)),
        out_shape=jax.ShapeDtypeStruct((BN, D), jnp.float32),
    )(nb2, h, Wg, Ws, bg)


def _softmax_lanes(s):
    m = jnp.max(s, axis=-1, keepdims=True)
    e = jnp.exp(s - m)
    return e / jnp.sum(e, axis=-1, keepdims=True)


def _score_body(hb_ref, w_ref, curr_ref):
    s = jnp.sum(hb_ref[...] * w_ref[...][None, None, :], axis=-1)
    curr_ref[...] = _softmax_lanes(s)


def _tc_score(hb, w):
    return pl.pallas_call(
        _score_body,
        out_shape=jax.ShapeDtypeStruct((B, N), jnp.float32),
    )(hb, w)


def _reform_body(hb_ref, watt_ref, qe_ref, ins_ref, wr_ref, br_ref, insn_ref):
    hb = hb_ref[...]
    satt = jnp.sum(hb * watt_ref[...][None, None, :], axis=-1)
    att = _softmax_lanes(satt)
    qe = qe_ref[...]
    global_rep = jnp.sum(att[:, :, None] * hb, axis=1)
    q_ent = jnp.sum(qe[:, :, None] * hb, axis=1)
    q_ent = q_ent / (jnp.sum(qe, axis=1, keepdims=True) + 1e-8)
    for i in range(NUM_INS):
        cat = jnp.concatenate([ins_ref[i], q_ent, global_rep], axis=1)
        insn_ref[i] = jnp.tanh(_mm(cat, wr_ref[i]) + br_ref[i][None, :])


def _tc_reform(hb, w_att_j, query_entities, ins, W_reform, b_reform):
    return pl.pallas_call(
        _reform_body,
        out_shape=jax.ShapeDtypeStruct((NUM_INS, B, D), jnp.float32),
    )(hb, w_att_j, query_entities, ins, W_reform, b_reform)


def _final_body(pd_ref, ans_ref, loss_ref, pred_ref):
    pd = pd_ref[...]
    ans = ans_ref[...]
    case_valid = (jnp.sum(ans, axis=1, keepdims=True) > 0).astype(jnp.float32)
    tp = -jnp.sum(ans * jnp.log(pd + 1e-8), axis=1, keepdims=True) * case_valid
    loss_ref[...] = (jnp.sum(tp) / B).reshape(1, 1)
    m = jnp.max(pd, axis=1, keepdims=True)
    idx = jax.lax.broadcasted_iota(jnp.int32, (B, N), 1)
    big = jnp.where(pd == m, idx, N)
    pred_ref[0, :] = jnp.min(big, axis=1)


def _tc_final(pred_dist, answer_dist):
    return pl.pallas_call(
        _final_body,
        out_shape=(
            jax.ShapeDtypeStruct((1, 1), jnp.float32),
            jax.ShapeDtypeStruct((1, B), jnp.int32),
        ),
    )(pred_dist, answer_dist)


def kernel(local_entity, query_entities, edge_head, edge_tail, edge_rel,
           query_text, seed_dist, answer_dist, word_emb, rel_emb, rel_emb_inv,
           W_rel, b_rel, W_type, b_type, ins_W, ins_b, W_gnn, W_self, b_gnn,
           w_score, w_att, W_reform, b_reform):
    edge_head = edge_head.astype(jnp.int32)
    edge_tail = edge_tail.astype(jnp.int32)
    edge_rel = edge_rel.astype(jnp.int32)

    rf, t_type = _tc_prep(rel_emb, rel_emb_inv, W_rel, b_rel)
    dst_th = jnp.concatenate([edge_tail, edge_head])
    dst_tt = jnp.concatenate([edge_tail, edge_tail])
    ones_bn = jnp.ones((BN,), jnp.float32)
    qidx = query_text.reshape(B * L).astype(jnp.int32)
    edge_combo = _tc_combo(edge_head, edge_rel)
    epack_th, erem_th = _pack_edges(edge_combo, dst_th, edge_head)
    epack_tt, erem_tt = _pack_edges(edge_combo, dst_tt, edge_head)
    agg_f = _scatter_sc(epack_th, erem_th, t_type, ones_bn)
    ones_tbl = jnp.ones((NC * B * NUM_REL, D), jnp.float32)
    dep1 = agg_f[0, 0] * 0.0
    deg_f = _scatter_sc(epack_th, erem_th, ones_tbl, ones_bn + dep1)
    qrows = _q_sc(qidx, word_emb)
    agg2 = agg_f.reshape(NC, BN, D)
    deg2 = deg_f.reshape(NC, BN, D)
    h = _tc_h0(agg2, deg2, W_type, b_type)

    ins, curr = _tc_ins0(qrows, ins_W, ins_b, seed_dist)
    curr = curr + deg_f[0, 0] * 0.0

    Wg_split = W_gnn.reshape(NUM_GNN, NUM_INS, D, D)
    for t in range(NUM_ITER):
        t2 = _tc_tables(rf, ins)
        t2_flat = t2.reshape(NUM_INS * B * NUM_REL, D)
        for j in range(NUM_GNN):
            nb_f = _scatter_sc(epack_tt, erem_tt, t2_flat, curr.reshape(BN))
            nb2 = nb_f.reshape(NC, BN, D)
            h = _tc_layer(nb2, h, Wg_split[j], W_self[j], b_gnn[j])
            curr = _tc_score(h.reshape(B, N, D), w_score[j])
        ins = _tc_reform(h.reshape(B, N, D), w_att[NUM_GNN - 1],
                         query_entities, ins, W_reform, b_reform)

    pred_dist = curr
    loss2, pred2 = _tc_final(pred_dist, answer_dist)
    loss = loss2.reshape(())
    pred = pred2.reshape(B)
    return (loss, pred, pred_dist)
